# Initial kernel scaffold; baseline (speedup 1.0000x reference)
import jax
import jax.numpy as jnp
from jax.experimental import pallas as pl


def kernel(pos):
    raise NotImplementedError("write your pallas kernel here")


# speedup vs baseline: 2.6895x; 2.6895x over previous
import jax
import jax.numpy as jnp
from jax.experimental import pallas as pl
from jax.experimental.pallas import tpu as pltpu

_K = 50
_BINS = 5


def _topk_body(r_blk, n, pos_blk_ref, posT_ref, sqr_ref, sqc_ref,
               nbr_ref, dist_ref, rmax_ref):
    i = pl.program_id(0)
    pos_r = pos_blk_ref[...]
    posT = posT_ref[...]
    sq_r = sqr_ref[...]
    sq_c = sqc_ref[...]
    dot = jax.lax.dot_general(pos_r, posT, (((1,), (0,)), ((), ())),
                              preferred_element_type=jnp.float32)
    d2 = jnp.maximum(sq_r + sq_c - 2.0 * dot, 0.0)
    col = jax.lax.broadcasted_iota(jnp.int32, (r_blk, n), 1)
    row_g = i * r_blk + jax.lax.broadcasted_iota(jnp.int32, (r_blk, n), 0)
    d2 = jnp.where(col == row_g, 1e10, d2)

    dx = posT[0:1, :] - pos_r[:, 0:1]
    dy = posT[1:2, :] - pos_r[:, 1:2]
    dz = posT[2:3, :] - pos_r[:, 2:3]
    d2t = dx * dx + dy * dy + dz * dz

    big = jnp.float32(3e38)
    kk = jax.lax.broadcasted_iota(jnp.int32, (r_blk, _K), 1)

    def body(k, carry):
        d2c, vals, idxs = carry
        m = jnp.min(d2c, axis=1, keepdims=True)
        cand = jnp.where(d2c == m, col, n)
        sel = jnp.min(cand, axis=1, keepdims=True)
        d2c = jnp.where(col == sel, big, d2c)
        tv = jnp.min(jnp.where(col == sel, d2t, big), axis=1, keepdims=True)
        vals = jnp.where(kk == k, tv, vals)
        idxs = jnp.where(kk == k, sel, idxs)
        return d2c, vals, idxs

    vals0 = jnp.zeros((r_blk, _K), jnp.float32)
    idxs0 = jnp.zeros((r_blk, _K), jnp.int32)
    _, vals, idxs = jax.lax.fori_loop(0, _K, body, (d2, vals0, idxs0))
    nbr_ref[...] = idxs
    dist = jnp.sqrt(vals + 1e-12)
    dist_ref[...] = dist
    rmax_ref[...] = jnp.max(dist, axis=1, keepdims=True)


def _rbf_body(cut_ref, dist_ref, out_ref):
    c = cut_ref[0]
    delta = c * 0.25
    sigma = delta + 1e-9
    inv = 1.0 / (2.0 * sigma * sigma)
    d = dist_ref[...]
    centers = jax.lax.broadcasted_iota(
        jnp.int32, (1, _BINS), 1).astype(jnp.float32) * delta
    out_ref[...] = jnp.exp(-((d - centers) ** 2) * inv)


def kernel(pos):
    n = pos.shape[0]
    r_blk = next(r for r in (200, 100, 40, 8, 1) if n % r == 0)
    posT = pos.T
    sq = jnp.sum(pos * pos, axis=1)

    nbr, dist, rmax = pl.pallas_call(
        lambda *refs: _topk_body(r_blk, n, *refs),
        grid=(n // r_blk,),
        in_specs=[
            pl.BlockSpec((r_blk, 3), lambda i: (i, 0)),
            pl.BlockSpec((3, n), lambda i: (0, 0)),
            pl.BlockSpec((r_blk, 1), lambda i: (i, 0)),
            pl.BlockSpec((1, n), lambda i: (0, 0)),
        ],
        out_specs=[
            pl.BlockSpec((r_blk, _K), lambda i: (i, 0)),
            pl.BlockSpec((r_blk, _K), lambda i: (i, 0)),
            pl.BlockSpec((r_blk, 1), lambda i: (i, 0)),
        ],
        out_shape=[
            jax.ShapeDtypeStruct((n, _K), jnp.int32),
            jax.ShapeDtypeStruct((n, _K), jnp.float32),
            jax.ShapeDtypeStruct((n, 1), jnp.float32),
        ],
    )(pos, posT, sq.reshape(n, 1), sq.reshape(1, n))

    cutoff = jnp.max(rmax).reshape(1)

    e = n * _K
    b_blk = next(b for b in (20000, 5000, 1000, 200, 50) if e % b == 0)
    ea_half = pl.pallas_call(
        _rbf_body,
        grid=(e // b_blk,),
        in_specs=[
            pl.BlockSpec(memory_space=pltpu.SMEM),
            pl.BlockSpec((b_blk, 1), lambda j: (j, 0)),
        ],
        out_specs=pl.BlockSpec((b_blk, _BINS), lambda j: (j, 0)),
        out_shape=jax.ShapeDtypeStruct((e, _BINS), jnp.float32),
    )(cutoff, dist.reshape(e, 1))

    src = nbr.reshape(-1)
    dst = jnp.broadcast_to(
        jnp.arange(n, dtype=jnp.int32)[:, None], (n, _K)).reshape(-1)
    edge_index = jnp.stack([
        jnp.concatenate([src, dst]),
        jnp.concatenate([dst, src]),
    ])
    edge_attr = jnp.concatenate([ea_half, ea_half], axis=0)
    return edge_index, edge_attr
